# baseline (speedup 1.0000x reference)
import functools

import jax
import jax.numpy as jnp
from jax import lax
from jax.experimental import pallas as pl
from jax.experimental.pallas import tpu as pltpu
from jax.experimental.pallas import tpu_sc as plsc

NC = 2
NS = 16
NW = NC * NS
CH = 64


def _mesh():
    return plsc.VectorSubcoreMesh(core_axis_name="c", subcore_axis_name="s")


NB = 5
FF = 3
GI = 3
SW = 2


@functools.lru_cache(maxsize=None)
def _sc_agg_kernel(n_pad, k, d):
    rt = n_pad // NS

    @functools.partial(
        pl.kernel,
        out_type=jax.ShapeDtypeStruct((NC, n_pad, d), jnp.float32),
        mesh=_mesh(),
        scratch_types=(
            [pltpu.VMEM((2, CH), jnp.int32) for _ in range(NB)]
            + [pltpu.VMEM((CH, d), jnp.float32) for _ in range(NB)]
            + [pltpu.VMEM_SHARED((n_pad, d), jnp.float32)]
            + [pltpu.SemaphoreType.DMA for _ in range(3 * NB)]
        ),
    )
    def body(proj_hbm, sdw_hbm, out_hbm, *scr):
        idxs = scr[0:NB]
        rows = scr[NB:2 * NB]
        agg_sh = scr[2 * NB]
        fs = scr[2 * NB + 1:2 * NB + 1 + NB]
        gs = scr[2 * NB + 1 + NB:2 * NB + 1 + 2 * NB]
        ss = scr[2 * NB + 1 + 2 * NB:2 * NB + 1 + 3 * NB]
        c = lax.axis_index("c")
        s = lax.axis_index("s")
        wid = s * NC + c
        zeros = jnp.zeros((16,), jnp.float32)

        @pl.loop(0, CH)
        def _zero_rows(i):
            for t in range(d // 16):
                rows[0][i, pl.ds(t * 16, 16)] = zeros

        base = s * rt
        for off in range(0, rt, CH):
            m = min(CH, rt - off)
            pltpu.sync_copy(rows[0].at[pl.ds(0, m)], agg_sh.at[pl.ds(base + off, m)])
        plsc.subcore_barrier()

        def fetch(j, p):
            pltpu.async_copy(sdw_hbm.at[wid, j], idxs[p], fs[p])

        def fwait(j, p):
            pltpu.make_async_copy(sdw_hbm.at[wid, j], idxs[p], fs[p]).wait()

        def gissue(j, p):
            del j
            pltpu.async_copy(proj_hbm.at[idxs[p].at[0]], rows[p], gs[p])

        def gwait(j, p):
            del j
            pltpu.make_async_copy(proj_hbm.at[idxs[p].at[0]], rows[p], gs[p]).wait()

        def sissue(j, p):
            del j
            pltpu.async_copy(rows[p], agg_sh.at[idxs[p].at[1]], ss[p], add=True)

        def swait(j, p):
            del j
            pltpu.make_async_copy(rows[p], agg_sh.at[idxs[p].at[1]], ss[p]).wait()

        def step(j, q, lo, hi_f, hi_g):
            if lo:
                swait(j - SW, (q - SW) % NB)
            if hi_f:
                fetch(j + FF, (q + FF) % NB)
            if hi_g:
                fwait(j + GI, (q + GI) % NB)
                gissue(j + GI, (q + GI) % NB)
            gwait(j, q)
            sissue(j, q)

        for m in range(FF):
            fetch(m, m)
        for m in range(GI):
            fwait(m, m)
            gissue(m, m)
        for j in range(NB):
            step(j, j, j >= SW, True, True)

        @pl.loop(NB, k - NB, step=NB)
        def _chunk(j):
            for q in range(NB):
                step(j + q, q, True, True, True)

        for j in range(k - NB, k):
            step(j, j % NB, True, j + FF < k, j + GI < k)
        for j in range(k - SW, k):
            swait(j, j % NB)

        plsc.subcore_barrier()
        pltpu.sync_copy(agg_sh.at[pl.ds(base, rt)], out_hbm.at[c, pl.ds(base, rt)])

    return body


def _tc_proj(h, w):
    n, d = h.shape
    r = 2000
    assert n % r == 0

    def body(h_ref, w_ref, o_ref):
        o_ref[...] = jnp.dot(h_ref[...], w_ref[...], preferred_element_type=jnp.float32)

    return pl.pallas_call(
        body,
        grid=(n // r,),
        in_specs=[
            pl.BlockSpec((r, d), lambda i: (i, 0)),
            pl.BlockSpec((d, d), lambda i: (0, 0)),
        ],
        out_specs=pl.BlockSpec((r, d), lambda i: (i, 0)),
        out_shape=jax.ShapeDtypeStruct((n, d), jnp.float32),
    )(h, w)


def _tc_combine(h, parts, cparts, et, wself, w1, b1, w2, b2, wnext):
    n, d = h.shape
    n_pad = parts.shape[1]
    cw = cparts.shape[2]
    r = 2000
    assert n % r == 0
    has_next = wnext is not None

    def body(h_ref, p_ref, c_ref, et_ref, ws_ref, w1_ref, b1_ref, w2_ref, b2_ref, *rest):
        if has_next:
            wn_ref, hn_ref, pn_ref = rest
        else:
            (hn_ref,) = rest
        hh = h_ref[...]
        agg = p_ref[0] + p_ref[1]
        cnt = c_ref[0] + c_ref[1]
        agg = agg + jnp.dot(cnt, et_ref[...], preferred_element_type=jnp.float32)
        x = agg + jnp.dot(hh, ws_ref[...], preferred_element_type=jnp.float32)
        x = jax.nn.gelu(jnp.dot(x, w1_ref[...], preferred_element_type=jnp.float32) + b1_ref[...])
        x = jnp.dot(x, w2_ref[...], preferred_element_type=jnp.float32) + b2_ref[...]
        hn = x + hh
        hn_ref[...] = hn
        if has_next:
            pn_ref[...] = jnp.dot(hn, wn_ref[...], preferred_element_type=jnp.float32)

    in_specs = [
        pl.BlockSpec((r, d), lambda i: (i, 0)),
        pl.BlockSpec((NC, r, d), lambda i: (0, i, 0)),
        pl.BlockSpec((NC, r, cw), lambda i: (0, i, 0)),
        pl.BlockSpec((cw, d), lambda i: (0, 0)),
        pl.BlockSpec((d, d), lambda i: (0, 0)),
        pl.BlockSpec((d, d), lambda i: (0, 0)),
        pl.BlockSpec((1, d), lambda i: (0, 0)),
        pl.BlockSpec((d, d), lambda i: (0, 0)),
        pl.BlockSpec((1, d), lambda i: (0, 0)),
    ]
    args = [h, parts, cparts, et, wself, w1, b1, w2, b2]
    out_shape = [jax.ShapeDtypeStruct((n, d), jnp.float32)]
    out_specs = [pl.BlockSpec((r, d), lambda i: (i, 0))]
    if has_next:
        in_specs.append(pl.BlockSpec((d, d), lambda i: (0, 0)))
        args.append(wnext)
        out_shape.append(jax.ShapeDtypeStruct((n, d), jnp.float32))
        out_specs.append(pl.BlockSpec((r, d), lambda i: (i, 0)))

    res = pl.pallas_call(
        body,
        grid=(n // r,),
        in_specs=in_specs,
        out_specs=out_specs,
        out_shape=out_shape,
    )(*args)
    return (res[0], res[1]) if has_next else (res[0], None)


def kernel(features, pe, edge_index, edge_type, W_self, W_msg, edge_tab, W1, b1, W2, b2):
    n, d = features.shape
    num_layers = W_self.shape[0]
    e = edge_index.shape[1]
    ne = edge_tab.shape[1]

    ew = -(-e // NW)
    k = -(-ew // CH)
    k = max(2 * NB, NB * (-(-k // NB)))
    ep = NW * k * CH
    n_pad = 128 * (-(-(n + 1) // 128))

    src = edge_index[0]
    dst = edge_index[1]
    padlen = ep - e
    srcw = jnp.concatenate([src, jnp.zeros((padlen,), jnp.int32)]).reshape(NW, k, CH)
    dstw = jnp.concatenate([dst, jnp.full((padlen,), n, jnp.int32)]).reshape(NW, k, CH)
    typw = jnp.concatenate([edge_type, jnp.zeros((padlen,), jnp.int32)]).reshape(NW, k, CH)
    sdw = jnp.stack([srcw, dstw], axis=2)

    et_pad = jnp.zeros((num_layers, d, d), jnp.float32).at[:, :ne, :].set(
        edge_tab.reshape(num_layers, ne, d))
    b1r = b1.reshape(num_layers, 1, d)
    b2r = b2.reshape(num_layers, 1, d)

    onehot = jnp.tile(jnp.eye(ne, d, dtype=jnp.float32), (CH, 1))
    tgw = typw + ne * jnp.arange(CH, dtype=jnp.int32)[None, None, :]
    tdw = jnp.stack([tgw, dstw], axis=2)
    cparts = _sc_agg_kernel(n_pad, k, d)(onehot, tdw)

    h = features
    proj = _tc_proj(h, W_msg[0, 0])
    for l in range(num_layers):
        parts = _sc_agg_kernel(n_pad, k, d)(proj, sdw)
        wnext = W_msg[l + 1, 0] if l + 1 < num_layers else None
        h, proj = _tc_combine(
            h, parts, cparts, et_pad[l], W_self[l, 0], W1[l], b1r[l], W2[l], b2r[l], wnext)
    return h


# speedup vs baseline: 1.0235x; 1.0235x over previous
import functools

import jax
import jax.numpy as jnp
from jax import lax
from jax.experimental import pallas as pl
from jax.experimental.pallas import tpu as pltpu
from jax.experimental.pallas import tpu_sc as plsc

NC = 2
NS = 16
NW = NC * NS
CH = 64


def _mesh():
    return plsc.VectorSubcoreMesh(core_axis_name="c", subcore_axis_name="s")


NB = 5
FF = 3
GI = 2
SW = 2


@functools.lru_cache(maxsize=None)
def _sc_agg_kernel(n_pad, k, d):
    rt = n_pad // NS

    @functools.partial(
        pl.kernel,
        out_type=jax.ShapeDtypeStruct((NC, n_pad, d), jnp.float32),
        mesh=_mesh(),
        scratch_types=(
            [pltpu.VMEM((2, CH), jnp.int32) for _ in range(NB)]
            + [pltpu.VMEM((CH, d), jnp.float32) for _ in range(NB)]
            + [pltpu.VMEM_SHARED((n_pad, d), jnp.float32)]
            + [pltpu.SemaphoreType.DMA for _ in range(3 * NB)]
        ),
    )
    def body(proj_hbm, sdw_hbm, out_hbm, *scr):
        idxs = scr[0:NB]
        rows = scr[NB:2 * NB]
        agg_sh = scr[2 * NB]
        fs = scr[2 * NB + 1:2 * NB + 1 + NB]
        gs = scr[2 * NB + 1 + NB:2 * NB + 1 + 2 * NB]
        ss = scr[2 * NB + 1 + 2 * NB:2 * NB + 1 + 3 * NB]
        c = lax.axis_index("c")
        s = lax.axis_index("s")
        wid = s * NC + c
        zeros = jnp.zeros((16,), jnp.float32)

        @pl.loop(0, CH)
        def _zero_rows(i):
            for t in range(d // 16):
                rows[0][i, pl.ds(t * 16, 16)] = zeros

        base = s * rt
        for off in range(0, rt, CH):
            m = min(CH, rt - off)
            pltpu.sync_copy(rows[0].at[pl.ds(0, m)], agg_sh.at[pl.ds(base + off, m)])
        plsc.subcore_barrier()

        def fetch(j, p):
            pltpu.async_copy(sdw_hbm.at[wid, j], idxs[p], fs[p])

        def fwait(j, p):
            pltpu.make_async_copy(sdw_hbm.at[wid, j], idxs[p], fs[p]).wait()

        def gissue(j, p):
            del j
            pltpu.async_copy(proj_hbm.at[idxs[p].at[0]], rows[p], gs[p])

        def gwait(j, p):
            del j
            pltpu.make_async_copy(proj_hbm.at[idxs[p].at[0]], rows[p], gs[p]).wait()

        def sissue(j, p):
            del j
            pltpu.async_copy(rows[p], agg_sh.at[idxs[p].at[1]], ss[p], add=True)

        def swait(j, p):
            del j
            pltpu.make_async_copy(rows[p], agg_sh.at[idxs[p].at[1]], ss[p]).wait()

        def step(j, q, lo, hi_f, hi_g):
            if lo:
                swait(j - SW, (q - SW) % NB)
            if hi_f:
                fetch(j + FF, (q + FF) % NB)
            if hi_g:
                fwait(j + GI, (q + GI) % NB)
                gissue(j + GI, (q + GI) % NB)
            gwait(j, q)
            sissue(j, q)

        for m in range(FF):
            fetch(m, m)
        for m in range(GI):
            fwait(m, m)
            gissue(m, m)
        for j in range(NB):
            step(j, j, j >= SW, True, True)

        @pl.loop(NB, k - NB, step=NB)
        def _chunk(j):
            for q in range(NB):
                step(j + q, q, True, True, True)

        for j in range(k - NB, k):
            step(j, j % NB, True, j + FF < k, j + GI < k)
        for j in range(k - SW, k):
            swait(j, j % NB)

        plsc.subcore_barrier()
        pltpu.sync_copy(agg_sh.at[pl.ds(base, rt)], out_hbm.at[c, pl.ds(base, rt)])

    return body


def _tc_proj(h, w):
    n, d = h.shape
    r = 2000
    assert n % r == 0

    def body(h_ref, w_ref, o_ref):
        o_ref[...] = jnp.dot(h_ref[...], w_ref[...], preferred_element_type=jnp.float32)

    return pl.pallas_call(
        body,
        grid=(n // r,),
        in_specs=[
            pl.BlockSpec((r, d), lambda i: (i, 0)),
            pl.BlockSpec((d, d), lambda i: (0, 0)),
        ],
        out_specs=pl.BlockSpec((r, d), lambda i: (i, 0)),
        out_shape=jax.ShapeDtypeStruct((n, d), jnp.float32),
    )(h, w)


def _tc_combine(h, parts, cparts, et, wself, w1, b1, w2, b2, wnext):
    n, d = h.shape
    n_pad = parts.shape[1]
    cw = cparts.shape[2]
    r = 2000
    assert n % r == 0
    has_next = wnext is not None

    def body(h_ref, p_ref, c_ref, et_ref, ws_ref, w1_ref, b1_ref, w2_ref, b2_ref, *rest):
        if has_next:
            wn_ref, hn_ref, pn_ref = rest
        else:
            (hn_ref,) = rest
        hh = h_ref[...]
        agg = p_ref[0] + p_ref[1]
        cnt = c_ref[0] + c_ref[1]
        agg = agg + jnp.dot(cnt, et_ref[...], preferred_element_type=jnp.float32)
        x = agg + jnp.dot(hh, ws_ref[...], preferred_element_type=jnp.float32)
        x = jax.nn.gelu(jnp.dot(x, w1_ref[...], preferred_element_type=jnp.float32) + b1_ref[...])
        x = jnp.dot(x, w2_ref[...], preferred_element_type=jnp.float32) + b2_ref[...]
        hn = x + hh
        hn_ref[...] = hn
        if has_next:
            pn_ref[...] = jnp.dot(hn, wn_ref[...], preferred_element_type=jnp.float32)

    in_specs = [
        pl.BlockSpec((r, d), lambda i: (i, 0)),
        pl.BlockSpec((NC, r, d), lambda i: (0, i, 0)),
        pl.BlockSpec((NC, r, cw), lambda i: (0, i, 0)),
        pl.BlockSpec((cw, d), lambda i: (0, 0)),
        pl.BlockSpec((d, d), lambda i: (0, 0)),
        pl.BlockSpec((d, d), lambda i: (0, 0)),
        pl.BlockSpec((1, d), lambda i: (0, 0)),
        pl.BlockSpec((d, d), lambda i: (0, 0)),
        pl.BlockSpec((1, d), lambda i: (0, 0)),
    ]
    args = [h, parts, cparts, et, wself, w1, b1, w2, b2]
    out_shape = [jax.ShapeDtypeStruct((n, d), jnp.float32)]
    out_specs = [pl.BlockSpec((r, d), lambda i: (i, 0))]
    if has_next:
        in_specs.append(pl.BlockSpec((d, d), lambda i: (0, 0)))
        args.append(wnext)
        out_shape.append(jax.ShapeDtypeStruct((n, d), jnp.float32))
        out_specs.append(pl.BlockSpec((r, d), lambda i: (i, 0)))

    res = pl.pallas_call(
        body,
        grid=(n // r,),
        in_specs=in_specs,
        out_specs=out_specs,
        out_shape=out_shape,
    )(*args)
    return (res[0], res[1]) if has_next else (res[0], None)


def kernel(features, pe, edge_index, edge_type, W_self, W_msg, edge_tab, W1, b1, W2, b2):
    n, d = features.shape
    num_layers = W_self.shape[0]
    e = edge_index.shape[1]
    ne = edge_tab.shape[1]

    ew = -(-e // NW)
    k = -(-ew // CH)
    k = max(2 * NB, NB * (-(-k // NB)))
    ep = NW * k * CH
    n_pad = 128 * (-(-(n + 1) // 128))

    src = edge_index[0]
    dst = edge_index[1]
    padlen = ep - e
    srcw = jnp.concatenate([src, jnp.zeros((padlen,), jnp.int32)]).reshape(NW, k, CH)
    dstw = jnp.concatenate([dst, jnp.full((padlen,), n, jnp.int32)]).reshape(NW, k, CH)
    typw = jnp.concatenate([edge_type, jnp.zeros((padlen,), jnp.int32)]).reshape(NW, k, CH)
    sdw = jnp.stack([srcw, dstw], axis=2)

    et_pad = jnp.zeros((num_layers, d, d), jnp.float32).at[:, :ne, :].set(
        edge_tab.reshape(num_layers, ne, d))
    b1r = b1.reshape(num_layers, 1, d)
    b2r = b2.reshape(num_layers, 1, d)

    onehot = jnp.tile(jnp.eye(ne, d, dtype=jnp.float32), (CH, 1))
    tgw = typw + ne * jnp.arange(CH, dtype=jnp.int32)[None, None, :]
    tdw = jnp.stack([tgw, dstw], axis=2)
    cparts = _sc_agg_kernel(n_pad, k, d)(onehot, tdw)

    h = features
    proj = _tc_proj(h, W_msg[0, 0])
    for l in range(num_layers):
        parts = _sc_agg_kernel(n_pad, k, d)(proj, sdw)
        wnext = W_msg[l + 1, 0] if l + 1 < num_layers else None
        h, proj = _tc_combine(
            h, parts, cparts, et_pad[l], W_self[l, 0], W1[l], b1r[l], W2[l], b2r[l], wnext)
    return h
